# baseline (speedup 1.0000x reference)
import functools

import jax
import jax.numpy as jnp
from jax import lax
from jax.experimental import pallas as pl
from jax.experimental.pallas import tpu as pltpu
from jax.experimental.pallas import tpu_sc as plsc

_SEP = 16
_EPS = 1e-12
_NW = 32


def _gather_rows(table, ids):
    nb = ids.shape[0] * ids.shape[1]
    d = table.shape[1]
    b_per_w = nb // _NW
    mesh = plsc.VectorSubcoreMesh(core_axis_name="c", subcore_axis_name="s")

    @functools.partial(
        pl.kernel,
        mesh=mesh,
        out_type=jax.ShapeDtypeStruct((nb, d), jnp.float32),
        scratch_types=[
            pltpu.VMEM((b_per_w,), jnp.int32),
            pltpu.VMEM((b_per_w, d), jnp.float32),
            pltpu.SemaphoreType.DMA,
        ],
        compiler_params=pltpu.CompilerParams(needs_layout_passes=False),
    )
    def k(table_hbm, idx_hbm, out_hbm, idx_v, rows_v, sem):
        wid = lax.axis_index("s") * 2 + lax.axis_index("c")
        base = wid * b_per_w
        l = idx_hbm.shape[1]
        pltpu.sync_copy(
            idx_hbm.at[base // l, pl.ds(base % l, b_per_w)], idx_v)
        lane = lax.iota(jnp.int32, 16)

        def chunk(c, carry):
            vals = idx_v[pl.ds(c * 16, 16)]
            copies = [pltpu.async_copy(
                table_hbm.at[pl.ds(jnp.sum(jnp.where(lane == j, vals, 0)), 1), :],
                rows_v.at[pl.ds(c * 16 + j, 1), :], sem)
                for j in range(16)]
            for cp in copies:
                cp.wait()
            return carry

        lax.fori_loop(0, b_per_w // 16, chunk, 0)
        pltpu.sync_copy(rows_v, out_hbm.at[pl.ds(base, b_per_w)])

    return k(table, ids)


def _dense_body(gx_ref, mask_ref, ln1w_ref, ln1b_ref, fcw_ref, fcb_ref,
                ln2w_ref, ln2b_ref, pe_ref, out_ref):
    b, l = mask_ref.shape
    nseg = pe_ref.shape[0]
    x = gx_ref[...]
    u = jnp.mean(x, axis=1, keepdims=True)
    xc = x - u
    v = jnp.mean(xc * xc, axis=1, keepdims=True)
    h = xc * lax.rsqrt(v + _EPS) * ln1w_ref[...] + ln1b_ref[...]
    y = lax.dot_general(h, fcw_ref[...], (((1,), (1,)), ((), ())),
                        preferred_element_type=jnp.float32,
                        precision=lax.Precision.DEFAULT)
    y = jnp.maximum(y + fcb_ref[...], 0.0)
    u2 = jnp.mean(y, axis=1, keepdims=True)
    yc = y - u2
    v2 = jnp.mean(yc * yc, axis=1, keepdims=True)
    z = yc * lax.rsqrt(v2 + _EPS) * ln2w_ref[...] + ln2b_ref[...]
    row = lax.broadcasted_iota(jnp.int32, (nseg, l), 0)
    col = lax.broadcasted_iota(jnp.int32, (nseg, l), 1)
    base = (col // _SEP == row) & (col % _SEP != _SEP - 1)
    for i in range(b):
        keep = base & (mask_ref[pl.ds(i, 1), :] != 1)
        pool = jnp.where(keep, 1.0 / (_SEP - 1), 0.0)
        seg = lax.dot_general(pool, z[i * l:(i + 1) * l], (((1,), (0,)), ((), ())),
                              preferred_element_type=jnp.float32,
                              precision=lax.Precision.DEFAULT)
        out_ref[i] = seg + pe_ref[...]


def _dense(g2, mask2, ln1w, ln1b, fcw, fcb, ln2w, ln2b, pe, nseg):
    b, l = mask2.shape
    wvec = fcw.shape[1]
    hid = fcw.shape[0]
    return pl.pallas_call(
        _dense_body,
        grid=(1,),
        in_specs=[
            pl.BlockSpec((b * l, wvec), lambda i: (0, 0)),
            pl.BlockSpec((b, l), lambda i: (0, 0)),
            pl.BlockSpec((wvec,), lambda i: (0,)),
            pl.BlockSpec((wvec,), lambda i: (0,)),
            pl.BlockSpec((hid, wvec), lambda i: (0, 0)),
            pl.BlockSpec((hid,), lambda i: (0,)),
            pl.BlockSpec((hid,), lambda i: (0,)),
            pl.BlockSpec((hid,), lambda i: (0,)),
            pl.BlockSpec((nseg, hid), lambda i: (0, 0)),
        ],
        out_specs=pl.BlockSpec((b, nseg, hid), lambda i: (0, 0, 0)),
        out_shape=jax.ShapeDtypeStruct((b, nseg, hid), jnp.float32),
    )(g2, mask2, ln1w, ln1b, fcw, fcb, ln2w, ln2b, pe)


def kernel(ingr_input_ids, ingr_sep_masks, emb_table, ln1_w, ln1_b,
           fc_W, fc_b, ln2_w, ln2_b, pe):
    b, l = ingr_input_ids.shape
    nseg = l // _SEP
    ids = ingr_input_ids.astype(jnp.int32)
    gathered = _gather_rows(emb_table.astype(jnp.float32), ids)
    return _dense(
        gathered,
        ingr_sep_masks.astype(jnp.int32),
        ln1_w, ln1_b, fc_W, fc_b, ln2_w, ln2_b, pe, nseg,
    )


# speedup vs baseline: 1.0424x; 1.0424x over previous
import functools

import jax
import jax.numpy as jnp
from jax import lax
from jax.experimental import pallas as pl
from jax.experimental.pallas import tpu as pltpu
from jax.experimental.pallas import tpu_sc as plsc

_SEP = 16
_EPS = 1e-12
_NW = 32


def _gather_rows(table, ids):
    nb = ids.shape[0] * ids.shape[1]
    d = table.shape[1]
    b_per_w = nb // _NW
    mesh = plsc.VectorSubcoreMesh(core_axis_name="c", subcore_axis_name="s")

    @functools.partial(
        pl.kernel,
        mesh=mesh,
        out_type=jax.ShapeDtypeStruct((nb, d), jnp.float32),
        scratch_types=[
            pltpu.VMEM((b_per_w,), jnp.int32),
            pltpu.VMEM((b_per_w, d), jnp.float32),
            pltpu.SemaphoreType.DMA,
        ],
        compiler_params=pltpu.CompilerParams(needs_layout_passes=False),
    )
    def k(table_hbm, idx_hbm, out_hbm, idx_v, rows_v, sem):
        wid = lax.axis_index("s") * 2 + lax.axis_index("c")
        base = wid * b_per_w
        l = idx_hbm.shape[1]
        pltpu.sync_copy(
            idx_hbm.at[base // l, pl.ds(base % l, b_per_w)], idx_v)
        lane = lax.iota(jnp.int32, 16)

        def fire(c):
            vals = idx_v[pl.ds(c * 16, 16)]
            return [pltpu.async_copy(
                table_hbm.at[pl.ds(jnp.sum(jnp.where(lane == j, vals, 0)), 1), :],
                rows_v.at[pl.ds(c * 16 + j, 1), :], sem)
                for j in range(16)]

        def chunk(c, carry):
            for cp in fire(c):
                cp.wait()
            return carry

        primed = fire(0)
        lax.fori_loop(1, b_per_w // 16, chunk, 0)
        for cp in primed:
            cp.wait()
        pltpu.sync_copy(rows_v, out_hbm.at[pl.ds(base, b_per_w)])

    return k(table, ids)


def _dense_body(gx_ref, mask_ref, ln1w_ref, ln1b_ref, fcw_ref, fcb_ref,
                ln2w_ref, ln2b_ref, pe_ref, out_ref):
    b, l = mask_ref.shape
    nseg = pe_ref.shape[0]
    x = gx_ref[...]
    u = jnp.mean(x, axis=1, keepdims=True)
    xc = x - u
    v = jnp.mean(xc * xc, axis=1, keepdims=True)
    h = xc * lax.rsqrt(v + _EPS) * ln1w_ref[...] + ln1b_ref[...]
    y = lax.dot_general(h, fcw_ref[...], (((1,), (1,)), ((), ())),
                        preferred_element_type=jnp.float32,
                        precision=lax.Precision.DEFAULT)
    y = jnp.maximum(y + fcb_ref[...], 0.0)
    u2 = jnp.mean(y, axis=1, keepdims=True)
    yc = y - u2
    v2 = jnp.mean(yc * yc, axis=1, keepdims=True)
    z = yc * lax.rsqrt(v2 + _EPS) * ln2w_ref[...] + ln2b_ref[...]
    row = lax.broadcasted_iota(jnp.int32, (nseg, l), 0)
    col = lax.broadcasted_iota(jnp.int32, (nseg, l), 1)
    base = (col // _SEP == row) & (col % _SEP != _SEP - 1)
    for i in range(b):
        keep = base & (mask_ref[pl.ds(i, 1), :] != 1)
        pool = jnp.where(keep, 1.0 / (_SEP - 1), 0.0)
        seg = lax.dot_general(pool, z[i * l:(i + 1) * l], (((1,), (0,)), ((), ())),
                              preferred_element_type=jnp.float32,
                              precision=lax.Precision.DEFAULT)
        out_ref[i] = seg + pe_ref[...]


def _dense(g2, mask2, ln1w, ln1b, fcw, fcb, ln2w, ln2b, pe, nseg):
    b, l = mask2.shape
    wvec = fcw.shape[1]
    hid = fcw.shape[0]
    return pl.pallas_call(
        _dense_body,
        grid=(1,),
        in_specs=[
            pl.BlockSpec((b * l, wvec), lambda i: (0, 0)),
            pl.BlockSpec((b, l), lambda i: (0, 0)),
            pl.BlockSpec((wvec,), lambda i: (0,)),
            pl.BlockSpec((wvec,), lambda i: (0,)),
            pl.BlockSpec((hid, wvec), lambda i: (0, 0)),
            pl.BlockSpec((hid,), lambda i: (0,)),
            pl.BlockSpec((hid,), lambda i: (0,)),
            pl.BlockSpec((hid,), lambda i: (0,)),
            pl.BlockSpec((nseg, hid), lambda i: (0, 0)),
        ],
        out_specs=pl.BlockSpec((b, nseg, hid), lambda i: (0, 0, 0)),
        out_shape=jax.ShapeDtypeStruct((b, nseg, hid), jnp.float32),
    )(g2, mask2, ln1w, ln1b, fcw, fcb, ln2w, ln2b, pe)


def kernel(ingr_input_ids, ingr_sep_masks, emb_table, ln1_w, ln1_b,
           fc_W, fc_b, ln2_w, ln2_b, pe):
    b, l = ingr_input_ids.shape
    nseg = l // _SEP
    ids = ingr_input_ids.astype(jnp.int32)
    gathered = _gather_rows(emb_table.astype(jnp.float32), ids)
    return _dense(
        gathered,
        ingr_sep_masks.astype(jnp.int32),
        ln1_w, ln1_b, fc_W, fc_b, ln2_w, ln2_b, pe, nseg,
    )
